# Initial kernel scaffold; baseline (speedup 1.0000x reference)
import jax
import jax.numpy as jnp
from jax.experimental import pallas as pl


def kernel(x, conv1_w, conv1_b, conv2_w, conv2_b, fc1_w, fc1_b, fc2_w, fc2_b):
    raise NotImplementedError("write your pallas kernel here")


# speedup vs baseline: 11.9736x; 11.9736x over previous
import functools

import jax
import jax.numpy as jnp
from jax.experimental import pallas as pl
from jax.experimental.pallas import tpu as pltpu

_K = 9
_N = 8
_S = 2
_HIGH = jax.lax.Precision.HIGHEST


def _convnn_body(nk, xf_ref, xs_ref, wk_ref, b_ref, out_ref):
    xf = xf_ref[...]
    xs = xs_ref[...]
    d = jax.lax.dot_general(
        xf.astype(jnp.bfloat16), xs.astype(jnp.bfloat16),
        (((2,), (2,)), ((0,), (0,))),
        preferred_element_type=jnp.float32)
    a2 = jnp.sum(xf * xf, axis=2)
    b2 = jnp.sum(xs * xs, axis=2)
    dist = a2[:, :, None] + b2[:, None, :] - 2.0 * d
    iota = jax.lax.broadcasted_iota(jnp.int32, dist.shape, 2)
    s_cand = dist.shape[2]
    acc = jnp.zeros(out_ref.shape, jnp.float32)
    for k in range(nk):
        cm = jnp.min(dist, axis=2, keepdims=True)
        idx = jnp.min(jnp.where(dist <= cm, iota, s_cand), axis=2, keepdims=True)
        hit = iota == idx
        oh = hit.astype(jnp.float32)
        dist = jnp.where(hit, jnp.inf, dist)
        yk = jax.lax.dot_general(
            xs.astype(jnp.bfloat16), wk_ref[k].astype(jnp.bfloat16),
            (((2,), (0,)), ((), ())),
            preferred_element_type=jnp.float32)
        acc = acc + jax.lax.dot_general(
            oh, yk, (((2,), (1,)), ((0,), (0,))),
            precision=_HIGH, preferred_element_type=jnp.float32)
    out_ref[...] = jnp.maximum(acc + b_ref[...], 0.0)


def _convnn_layer(xf, xs, w, b, bb):
    B, M, C2 = xf.shape
    S = xs.shape[1]
    Cout = w.shape[0]
    wk = jnp.transpose(w, (2, 1, 0))
    bias = b.reshape(1, 1, Cout)
    return pl.pallas_call(
        functools.partial(_convnn_body, _K),
        grid=(B // bb,),
        in_specs=[
            pl.BlockSpec((bb, M, C2), lambda i: (i, 0, 0)),
            pl.BlockSpec((bb, S, C2), lambda i: (i, 0, 0)),
            pl.BlockSpec((_K, C2, Cout), lambda i: (0, 0, 0)),
            pl.BlockSpec((1, 1, Cout), lambda i: (0, 0, 0)),
        ],
        out_specs=pl.BlockSpec((bb, M, Cout), lambda i: (i, 0, 0)),
        out_shape=jax.ShapeDtypeStruct((B, M, Cout), jnp.float32),
    )(xf, xs, wk, bias)


def _fc_body(nsteps, a_ref, w1_ref, b1_ref, w2_ref, b2_ref, out_ref, acc_ref):
    step = pl.program_id(0)
    part = jax.lax.dot_general(
        a_ref[...].astype(jnp.bfloat16), w1_ref[...].astype(jnp.bfloat16),
        (((1,), (1,)), ((), ())),
        preferred_element_type=jnp.float32)

    @pl.when(step == 0)
    def _():
        acc_ref[...] = part

    @pl.when(step > 0)
    def _():
        acc_ref[...] = acc_ref[...] + part

    @pl.when(step == nsteps - 1)
    def _():
        h = jnp.maximum(acc_ref[...] + b1_ref[...], 0.0)
        out_ref[...] = jax.lax.dot_general(
            h.astype(jnp.bfloat16), w2_ref[...].astype(jnp.bfloat16),
            (((1,), (1,)), ((), ())),
            preferred_element_type=jnp.float32) + b2_ref[...]


def _fc_head(h, fc1_w, fc1_b, fc2_w, fc2_b, kb):
    B, K = h.shape
    H1 = fc1_w.shape[0]
    C10 = fc2_w.shape[0]
    Cp = 128
    w2p = jnp.zeros((Cp, H1), jnp.float32).at[:C10].set(fc2_w)
    b2p = jnp.zeros((1, Cp), jnp.float32).at[0, :C10].set(fc2_b)
    nsteps = K // kb
    out = pl.pallas_call(
        functools.partial(_fc_body, nsteps),
        grid=(nsteps,),
        in_specs=[
            pl.BlockSpec((B, kb), lambda i: (0, i)),
            pl.BlockSpec((H1, kb), lambda i: (0, i)),
            pl.BlockSpec((1, H1), lambda i: (0, 0)),
            pl.BlockSpec((Cp, H1), lambda i: (0, 0)),
            pl.BlockSpec((1, Cp), lambda i: (0, 0)),
        ],
        out_specs=pl.BlockSpec((B, Cp), lambda i: (0, 0)),
        out_shape=jax.ShapeDtypeStruct((B, Cp), jnp.float32),
        scratch_shapes=[pltpu.VMEM((B, H1), jnp.float32)],
    )(h, fc1_w, fc1_b.reshape(1, H1), w2p, b2p)
    return out[:, :C10]


def _coords(h, w, dtype):
    yy = jnp.linspace(-1.0, 1.0, h)
    xx = jnp.linspace(-1.0, 1.0, w)
    gy, gx = jnp.meshgrid(yy, xx, indexing="ij")
    return jnp.stack([gy, gx], axis=-1).reshape(h * w, 2).astype(dtype)


def _sample_idx(h, w, n):
    import numpy as np
    ri = np.round(np.linspace(0.0, h - 1, n)).astype(np.int32)
    ci = np.round(np.linspace(0.0, w - 1, n)).astype(np.int32)
    return (ri[:, None] * w + ci[None, :]).reshape(-1)


def kernel(x, conv1_w, conv1_b, conv2_w, conv2_b, fc1_w, fc1_b, fc2_w, fc2_b):
    B = x.shape[0]
    xu = x.reshape(B, 3, 16, 2, 16, 2).transpose(0, 1, 3, 5, 2, 4).reshape(B, 12, 256)
    xf1 = jnp.concatenate(
        [xu.transpose(0, 2, 1), jnp.broadcast_to(_coords(16, 16, x.dtype)[None], (B, 256, 2))],
        axis=2)
    sidx = _sample_idx(16, 16, _N)
    xs1 = xf1[:, sidx, :]
    o1 = _convnn_layer(xf1, xs1, conv1_w, conv1_b, bb=8)

    xf2 = jnp.concatenate(
        [o1, jnp.broadcast_to(_coords(16, 16, x.dtype)[None], (B, 256, 2))], axis=2)
    xs2 = xf2[:, sidx, :]
    o2 = _convnn_layer(xf2, xs2, conv2_w, conv2_b, bb=8)

    h2 = (o2.reshape(B, 16, 16, 32, 2, 2)
             .transpose(0, 3, 1, 4, 2, 5)
             .reshape(B, 32768))
    return _fc_head(h2, fc1_w, fc1_b, fc2_w, fc2_b, kb=2048)
